# Initial kernel scaffold; baseline (speedup 1.0000x reference)
import jax
import jax.numpy as jnp
from jax.experimental import pallas as pl


def kernel(x, edge_index, Wl0, bl0, Wr0, Wl1, bl1, Wr1, Wl2, bl2, Wr2, gamma0, beta0, gamma1, beta1):
    raise NotImplementedError("write your pallas kernel here")


# speedup vs baseline: 4.3604x; 4.3604x over previous
import functools

import jax
import jax.numpy as jnp
from jax import lax
from jax.experimental import pallas as pl
from jax.experimental.pallas import tpu as pltpu
from jax.experimental.pallas import tpu_sc as plsc

N = 10000
E = 320000
D = 128
EPS = 1e-5

NC = 2
NS = 16
NW = NC * NS

CHUNK = 128
NCHUNKS = 79
EPW = CHUNK * NCHUNKS
E_PAD = EPW * NW
N_PAD = 10240
RPT = N_PAD // NS

BLK = 2048
NBLK = 5


def _make_segsum(with_counts):
    out_type = [jax.ShapeDtypeStruct((NC, N_PAD, D), jnp.float32)]
    if with_counts:
        out_type.append(jax.ShapeDtypeStruct((NC, N_PAD), jnp.float32))

    scratch = dict(
        idx_s=pltpu.VMEM((1, CHUNK), jnp.int32),
        idx_d=pltpu.VMEM((1, CHUNK), jnp.int32),
        rows=pltpu.VMEM((CHUNK, D), jnp.float32),
        ones_v=pltpu.VMEM((CHUNK,), jnp.float32),
        zcnt_v=pltpu.VMEM((RPT,), jnp.float32),
        acc_sh=pltpu.VMEM_SHARED((N_PAD, D), jnp.float32),
        cnt_sh=pltpu.VMEM_SHARED((N_PAD,), jnp.float32),
        sem=pltpu.SemaphoreType.DMA,
    )
    mesh = plsc.VectorSubcoreMesh(
        core_axis_name="c", subcore_axis_name="s", num_cores=NC, num_subcores=NS
    )

    @functools.partial(
        pl.kernel,
        out_type=out_type,
        mesh=mesh,
        scratch_types=scratch,
        name="sc_segsum_cnt" if with_counts else "sc_segsum",
    )
    def segsum(h_hbm, srci_hbm, dsti_hbm, zrow_hbm, zcnt_hbm, ones_hbm,
               out_hbm, *rest, idx_s, idx_d, rows, ones_v, zcnt_v, acc_sh,
               cnt_sh, sem):
        cnt_hbm = rest[0] if with_counts else None
        cid = lax.axis_index("c")
        sid = lax.axis_index("s")
        wid = cid * NS + sid

        pltpu.sync_copy(zrow_hbm, rows)
        for k in range(RPT // CHUNK):
            pltpu.sync_copy(rows, acc_sh.at[pl.ds(sid * RPT + k * CHUNK, CHUNK)])
        if with_counts:
            pltpu.sync_copy(ones_hbm, ones_v)
            pltpu.sync_copy(zcnt_hbm, zcnt_v)
            pltpu.sync_copy(zcnt_v, cnt_sh.at[pl.ds(sid * RPT, RPT)])
        plsc.subcore_barrier()

        base = wid * NCHUNKS

        def body(j, carry):
            pltpu.sync_copy(srci_hbm.at[base + j], idx_s.at[0])
            pltpu.sync_copy(dsti_hbm.at[base + j], idx_d.at[0])
            pltpu.async_copy(h_hbm.at[idx_s.at[0]], rows, sem).wait()
            pltpu.sync_copy(rows, acc_sh.at[idx_d.at[0]], add=True)
            if with_counts:
                pltpu.sync_copy(ones_v, cnt_sh.at[idx_d.at[0]], add=True)
            return carry

        lax.fori_loop(0, NCHUNKS, body, 0)
        plsc.subcore_barrier()

        pltpu.sync_copy(acc_sh.at[pl.ds(sid * RPT, RPT)],
                        out_hbm.at[cid].at[pl.ds(sid * RPT, RPT)])
        if with_counts:
            pltpu.sync_copy(cnt_sh.at[pl.ds(sid * RPT, RPT)],
                            cnt_hbm.at[cid].at[pl.ds(sid * RPT, RPT)])

    return segsum


_segsum_cnt = _make_segsum(True)
_segsum = _make_segsum(False)


def _linear_body(with_stats, P_ref, cnt_ref, h_ref, Wl_ref, bl_ref, Wr_ref,
                 out_ref, *stats):
    i = pl.program_id(0)
    c = cnt_ref[0] + cnt_ref[1]
    inv = 1.0 / jnp.clip(c, 1.0, None)
    mean = (P_ref[0] + P_ref[1]) * inv
    out = (
        lax.dot_general(mean, Wl_ref[...], (((1,), (1,)), ((), ())),
                        preferred_element_type=jnp.float32,
                        precision=lax.Precision.HIGHEST)
        + bl_ref[...][None, :]
        + lax.dot_general(h_ref[...], Wr_ref[...], (((1,), (1,)), ((), ())),
                          preferred_element_type=jnp.float32,
                          precision=lax.Precision.HIGHEST)
    )
    out_ref[...] = out
    if with_stats:
        stats_ref = stats[0]

        @pl.when(i == 0)
        def _():
            stats_ref[...] = jnp.zeros_like(stats_ref)

        row = i * BLK + lax.broadcasted_iota(jnp.int32, (BLK, D), 0)
        v = jnp.where(row < N, out, 0.0)
        stats_ref[0, :] += jnp.sum(v, axis=0)
        stats_ref[1, :] += jnp.sum(v * v, axis=0)


def _make_linear(with_stats):
    out_shape = [jax.ShapeDtypeStruct((N, D), jnp.float32)]
    out_specs = [pl.BlockSpec((BLK, D), lambda i: (i, 0))]
    if with_stats:
        out_shape.append(jax.ShapeDtypeStruct((8, D), jnp.float32))
        out_specs.append(pl.BlockSpec((8, D), lambda i: (0, 0)))
    return pl.pallas_call(
        functools.partial(_linear_body, with_stats),
        grid=(NBLK,),
        in_specs=[
            pl.BlockSpec((NC, BLK, D), lambda i: (0, i, 0)),
            pl.BlockSpec((NC, BLK, 1), lambda i: (0, i, 0)),
            pl.BlockSpec((BLK, D), lambda i: (i, 0)),
            pl.BlockSpec((D, D), lambda i: (0, 0)),
            pl.BlockSpec((D,), lambda i: (0,)),
            pl.BlockSpec((D, D), lambda i: (0, 0)),
        ],
        out_specs=out_specs,
        out_shape=out_shape,
        name="tc_linear_stats" if with_stats else "tc_linear",
    )


_linear_stats = _make_linear(True)
_linear_plain = _make_linear(False)


def _bn_body(stats_ref, gamma_ref, beta_ref, h_ref, out_ref, sc_ref):
    i = pl.program_id(0)

    @pl.when(i == 0)
    def _():
        mu = stats_ref[0] / N
        var = stats_ref[1] / N - mu * mu
        scale = gamma_ref[...] * lax.rsqrt(var + EPS)
        sc_ref[0, :] = scale
        sc_ref[1, :] = beta_ref[...] - mu * scale

    out_ref[...] = jnp.maximum(
        h_ref[...] * sc_ref[0, :][None, :] + sc_ref[1, :][None, :], 0.0)


_bn_relu = pl.pallas_call(
    _bn_body,
    grid=(NBLK,),
    in_specs=[
        pl.BlockSpec((8, D), lambda i: (0, 0)),
        pl.BlockSpec((D,), lambda i: (0,)),
        pl.BlockSpec((D,), lambda i: (0,)),
        pl.BlockSpec((BLK, D), lambda i: (i, 0)),
    ],
    out_specs=pl.BlockSpec((BLK, D), lambda i: (i, 0)),
    out_shape=jax.ShapeDtypeStruct((N, D), jnp.float32),
    scratch_shapes=[pltpu.VMEM((8, D), jnp.float32)],
    name="tc_bn_relu",
)


def kernel(x, edge_index, Wl0, bl0, Wr0, Wl1, bl1, Wr1, Wl2, bl2, Wr2,
           gamma0, beta0, gamma1, beta1):
    src = edge_index[0]
    dst = edge_index[1]
    pad = E_PAD - E
    srcp = jnp.concatenate([src, jnp.zeros((pad,), jnp.int32)])
    dstp = jnp.concatenate([dst, jnp.full((pad,), N, jnp.int32)])
    srcp = srcp.reshape(E_PAD // CHUNK, CHUNK)
    dstp = dstp.reshape(E_PAD // CHUNK, CHUNK)

    zrow = jnp.zeros((CHUNK, D), jnp.float32)
    zcnt = jnp.zeros((RPT,), jnp.float32)
    ones = jnp.ones((CHUNK,), jnp.float32)

    P, cnt = _segsum_cnt(x, srcp, dstp, zrow, zcnt, ones)
    cnt3 = cnt.reshape(NC, N_PAD, 1)

    h, stats = _linear_stats(P, cnt3, x, Wl0, bl0, Wr0)
    h = _bn_relu(stats, gamma0, beta0, h)

    [P] = _segsum(h, srcp, dstp, zrow, zcnt, ones)
    h, stats = _linear_stats(P, cnt3, h, Wl1, bl1, Wr1)
    h = _bn_relu(stats, gamma1, beta1, h)

    [P] = _segsum(h, srcp, dstp, zrow, zcnt, ones)
    [h] = _linear_plain(P, cnt3, h, Wl2, bl2, Wr2)
    return h
